# baseline (speedup 1.0000x reference)
import functools

import jax
import jax.numpy as jnp
import numpy as np
from jax.experimental import pallas as pl
from jax.experimental.pallas import tpu as pltpu

_TINY = np.float32(1.1754943508222875e-38)


def _rotl(x, r):
    return (x << jnp.uint32(r)) | (x >> jnp.uint32(32 - r))


def _round4(x0, x1, rots):
    for r in rots:
        x0 = x0 + x1
        x1 = _rotl(x1, r) ^ x0
    return x0, x1


def _threefry_bits(x_lo):
    ks1 = jnp.uint32(42)
    ks2 = jnp.uint32(0x1BD11BDA ^ 42)
    rot_a = (13, 15, 26, 6)
    rot_b = (17, 29, 16, 24)

    x1 = x_lo + ks1
    x0 = x1
    x1 = _rotl(x1, 13) ^ x0
    x0, x1 = _round4(x0, x1, rot_a[1:])
    x0 = x0 + ks1
    x1 = x1 + (ks2 + jnp.uint32(1))
    x0, x1 = _round4(x0, x1, rot_b)
    x0 = x0 + ks2
    x1 = x1 + jnp.uint32(2)
    x0, x1 = _round4(x0, x1, rot_a)
    x1 = x1 + (ks1 + jnp.uint32(3))
    x0, x1 = _round4(x0, x1, rot_b)
    x0 = x0 + ks1
    x1 = x1 + (ks2 + jnp.uint32(4))
    x0, x1 = _round4(x0, x1, rot_a)
    x0 = x0 + ks2
    x1 = x1 + jnp.uint32(5)
    return x0 ^ x1


def _body(vocab, n_chunks, lg_ref, out_ref, mx_ref, ix_ref):
    k = pl.program_id(1)
    rblk = pl.program_id(0)
    nb, nc = lg_ref.shape

    row = jax.lax.broadcasted_iota(jnp.uint32, (nb, nc), 0)
    col = jax.lax.broadcasted_iota(jnp.uint32, (nb, nc), 1)
    row0 = jnp.uint32(rblk) * jnp.uint32(nb)
    col0 = jnp.uint32(k) * jnp.uint32(nc)
    gcol_u = col + col0
    x_lo = (row + row0) * jnp.uint32(vocab) + gcol_u

    bits = _threefry_bits(x_lo)
    fb = (bits >> jnp.uint32(9)) | jnp.uint32(0x3F800000)
    u = jax.lax.bitcast_convert_type(fb, jnp.float32) - jnp.float32(1.0)
    uni = jnp.maximum(u, _TINY)
    g = -jnp.log(-jnp.log(uni))
    score = g + lg_ref[...]
    gcol = gcol_u.astype(jnp.int32)

    padded = vocab % nc != 0
    last = k == n_chunks - 1

    def _masked(s):
        return jnp.where(gcol < vocab, s, -jnp.inf) if padded else s

    def _carry(s):
        mx = mx_ref[...]
        better = s > mx
        mx_ref[...] = jnp.where(better, s, mx)
        ix_ref[...] = jnp.where(better, gcol, ix_ref[...])

    def _emit():
        mx = mx_ref[...]
        m = jnp.max(mx, axis=1, keepdims=True)
        cand = jnp.where(mx == m, ix_ref[...], jnp.int32(0x7FFFFFFF))
        out_ref[...] = jnp.min(cand, axis=1, keepdims=True)

    if n_chunks == 1:
        mx_ref[...] = _masked(score)
        ix_ref[...] = gcol
        _emit()
    else:
        @pl.when(k == 0)
        def _init():
            mx_ref[...] = score
            ix_ref[...] = gcol

        @pl.when(jnp.logical_and(k > 0, jnp.logical_not(last)))
        def _mid():
            _carry(score)

        @pl.when(last)
        def _fin():
            _carry(_masked(score))
            _emit()


def kernel(logits):
    batch, vocab = logits.shape
    row_blk = 16
    chunk = 16384
    n_rblk = pl.cdiv(batch, row_blk)
    n_chunks = pl.cdiv(vocab, chunk)

    out = pl.pallas_call(
        functools.partial(_body, vocab, n_chunks),
        grid=(n_rblk, n_chunks),
        in_specs=[pl.BlockSpec((row_blk, chunk), lambda r, k: (r, k))],
        out_specs=pl.BlockSpec((row_blk, 1), lambda r, k: (r, 0)),
        out_shape=jax.ShapeDtypeStruct((batch, 1), jnp.int32),
        scratch_shapes=[
            pltpu.VMEM((row_blk, chunk), jnp.float32),
            pltpu.VMEM((row_blk, chunk), jnp.int32),
        ],
        compiler_params=pltpu.CompilerParams(
            dimension_semantics=("parallel", "arbitrary"),
        ),
    )(logits)
    return out.reshape(batch)


# speedup vs baseline: 1.5449x; 1.5449x over previous
import functools

import jax
import jax.numpy as jnp
import numpy as np
from jax.experimental import pallas as pl
from jax.experimental.pallas import tpu as pltpu

_TINY = np.float32(1.1754943508222875e-38)


def _rotl(x, r):
    return (x << jnp.uint32(r)) | (x >> jnp.uint32(32 - r))


def _round4(x0, x1, rots):
    for r in rots:
        x0 = x0 + x1
        x1 = _rotl(x1, r) ^ x0
    return x0, x1


def _threefry_bits(x_lo):
    ks1 = jnp.uint32(42)
    ks2 = jnp.uint32(0x1BD11BDA ^ 42)
    rot_a = (13, 15, 26, 6)
    rot_b = (17, 29, 16, 24)

    x1 = x_lo + ks1
    x0 = x1
    x1 = _rotl(x1, 13) ^ x0
    x0, x1 = _round4(x0, x1, rot_a[1:])
    x0 = x0 + ks1
    x1 = x1 + (ks2 + jnp.uint32(1))
    x0, x1 = _round4(x0, x1, rot_b)
    x0 = x0 + ks2
    x1 = x1 + jnp.uint32(2)
    x0, x1 = _round4(x0, x1, rot_a)
    x1 = x1 + (ks1 + jnp.uint32(3))
    x0, x1 = _round4(x0, x1, rot_b)
    x0 = x0 + ks1
    x1 = x1 + (ks2 + jnp.uint32(4))
    x0, x1 = _round4(x0, x1, rot_a)
    x0 = x0 + ks2
    x1 = x1 + jnp.uint32(5)
    return x0 ^ x1


def _body(vocab, n_chunks, width, lg_ref, out_ref, mx_ref, ix_ref):
    k = pl.program_id(1)
    rblk = pl.program_id(0)
    nb, chunk = lg_ref.shape
    n_sub = chunk // width

    last = k == n_chunks - 1
    limit = jnp.where(last, jnp.int32(vocab), jnp.int32(0x7FFFFFFF))

    row = jax.lax.broadcasted_iota(jnp.uint32, (nb, width), 0)
    colw = jax.lax.broadcasted_iota(jnp.uint32, (nb, width), 1)
    row0 = jnp.uint32(rblk) * jnp.uint32(nb)
    rowterm = (row + row0) * jnp.uint32(vocab)
    chunk0 = jnp.uint32(k) * jnp.uint32(chunk)

    def _sub(i, carry):
        mx, ix = carry
        base = chunk0 + jnp.uint32(i) * jnp.uint32(width)
        gcol_u = colw + base
        bits = _threefry_bits(rowterm + gcol_u)
        fb = (bits >> jnp.uint32(9)) | jnp.uint32(0x3F800000)
        u = jax.lax.bitcast_convert_type(fb, jnp.float32) - jnp.float32(1.0)
        uni = jnp.maximum(u, _TINY)
        g = -jnp.log(-jnp.log(uni))
        s = g + lg_ref[:, pl.ds(i * width, width)]
        gcol = gcol_u.astype(jnp.int32)
        s = jnp.where(gcol < limit, s, -jnp.inf)
        better = s > mx
        mx = jnp.where(better, s, mx)
        ix = jnp.where(better, gcol, ix)
        return mx, ix

    mx0 = jnp.full((nb, width), -jnp.inf, jnp.float32)
    ix0 = jnp.zeros((nb, width), jnp.int32)
    mx, ix = jax.lax.fori_loop(0, n_sub, _sub, (mx0, ix0))

    @pl.when(k == 0)
    def _init():
        mx_ref[...] = mx
        ix_ref[...] = ix

    @pl.when(k > 0)
    def _merge():
        pmx = mx_ref[...]
        better = mx > pmx
        mx_ref[...] = jnp.where(better, mx, pmx)
        ix_ref[...] = jnp.where(better, ix, ix_ref[...])

    @pl.when(last)
    def _emit():
        fmx = mx_ref[...]
        m = jnp.max(fmx, axis=1, keepdims=True)
        cand = jnp.where(fmx == m, ix_ref[...], jnp.int32(0x7FFFFFFF))
        out_ref[...] = jnp.min(cand, axis=1, keepdims=True)


def kernel(logits):
    batch, vocab = logits.shape
    row_blk = 16
    chunk = 16384
    width = 512
    n_rblk = pl.cdiv(batch, row_blk)
    n_chunks = pl.cdiv(vocab, chunk)

    out = pl.pallas_call(
        functools.partial(_body, vocab, n_chunks, width),
        grid=(n_rblk, n_chunks),
        in_specs=[pl.BlockSpec((row_blk, chunk), lambda r, k: (r, k))],
        out_specs=pl.BlockSpec((row_blk, 1), lambda r, k: (r, 0)),
        out_shape=jax.ShapeDtypeStruct((batch, 1), jnp.int32),
        scratch_shapes=[
            pltpu.VMEM((row_blk, width), jnp.float32),
            pltpu.VMEM((row_blk, width), jnp.int32),
        ],
        compiler_params=pltpu.CompilerParams(
            dimension_semantics=("parallel", "arbitrary"),
        ),
    )(logits)
    return out.reshape(batch)


# speedup vs baseline: 1.7327x; 1.1216x over previous
import functools

import jax
import jax.numpy as jnp
import numpy as np
from jax.experimental import pallas as pl
from jax.experimental.pallas import tpu as pltpu

_TINY = np.float32(1.1754943508222875e-38)


def _rotl(x, r):
    return (x << jnp.uint32(r)) | (x >> jnp.uint32(32 - r))


def _round4(x0, x1, rots):
    for r in rots:
        x0 = x0 + x1
        x1 = _rotl(x1, r) ^ x0
    return x0, x1


def _threefry_bits(x_lo):
    ks1 = jnp.uint32(42)
    ks2 = jnp.uint32(0x1BD11BDA ^ 42)
    rot_a = (13, 15, 26, 6)
    rot_b = (17, 29, 16, 24)

    x1 = x_lo + ks1
    x0 = x1
    x1 = _rotl(x1, 13) ^ x0
    x0, x1 = _round4(x0, x1, rot_a[1:])
    x0 = x0 + ks1
    x1 = x1 + (ks2 + jnp.uint32(1))
    x0, x1 = _round4(x0, x1, rot_b)
    x0 = x0 + ks2
    x1 = x1 + jnp.uint32(2)
    x0, x1 = _round4(x0, x1, rot_a)
    x1 = x1 + (ks1 + jnp.uint32(3))
    x0, x1 = _round4(x0, x1, rot_b)
    x0 = x0 + ks1
    x1 = x1 + (ks2 + jnp.uint32(4))
    x0, x1 = _round4(x0, x1, rot_a)
    x0 = x0 + ks2
    x1 = x1 + jnp.uint32(5)
    return x0 ^ x1


def _body(vocab, n_chunks, width, lg_ref, out_ref, mx_ref, ix_ref):
    k = pl.program_id(1)
    rblk = pl.program_id(0)
    nb, chunk = lg_ref.shape
    n_sub = chunk // width

    last = k == n_chunks - 1
    limit = jnp.where(last, jnp.int32(vocab), jnp.int32(0x7FFFFFFF))

    row = jax.lax.broadcasted_iota(jnp.uint32, (nb, width), 0)
    colw = jax.lax.broadcasted_iota(jnp.uint32, (nb, width), 1)
    row0 = jnp.uint32(rblk) * jnp.uint32(nb)
    rowterm = (row + row0) * jnp.uint32(vocab)
    chunk0 = jnp.uint32(k) * jnp.uint32(chunk)

    def _sub(i, carry):
        mx, ix = carry
        base = chunk0 + jnp.uint32(i) * jnp.uint32(width)
        gcol_u = colw + base
        bits = _threefry_bits(rowterm + gcol_u)
        fb = (bits >> jnp.uint32(9)) | jnp.uint32(0x3F800000)
        u = jax.lax.bitcast_convert_type(fb, jnp.float32) - jnp.float32(1.0)
        uni = jnp.maximum(u, _TINY)
        g = -jnp.log(-jnp.log(uni))
        s = g + lg_ref[:, pl.ds(i * width, width)]
        gcol = gcol_u.astype(jnp.int32)
        s = jnp.where(gcol < limit, s, -jnp.inf)
        better = s > mx
        mx = jnp.where(better, s, mx)
        ix = jnp.where(better, gcol, ix)
        return mx, ix

    mx0 = jnp.full((nb, width), -jnp.inf, jnp.float32)
    ix0 = jnp.zeros((nb, width), jnp.int32)
    mx, ix = jax.lax.fori_loop(0, n_sub, _sub, (mx0, ix0), unroll=32)

    @pl.when(k == 0)
    def _init():
        mx_ref[...] = mx
        ix_ref[...] = ix

    @pl.when(k > 0)
    def _merge():
        pmx = mx_ref[...]
        better = mx > pmx
        mx_ref[...] = jnp.where(better, mx, pmx)
        ix_ref[...] = jnp.where(better, ix, ix_ref[...])

    @pl.when(last)
    def _emit():
        fmx = mx_ref[...]
        m = jnp.max(fmx, axis=1, keepdims=True)
        cand = jnp.where(fmx == m, ix_ref[...], jnp.int32(0x7FFFFFFF))
        out_ref[...] = jnp.min(cand, axis=1, keepdims=True)


def kernel(logits):
    batch, vocab = logits.shape
    row_blk = 16
    chunk = 16384
    width = 512
    n_rblk = pl.cdiv(batch, row_blk)
    n_chunks = pl.cdiv(vocab, chunk)

    out = pl.pallas_call(
        functools.partial(_body, vocab, n_chunks, width),
        grid=(n_rblk, n_chunks),
        in_specs=[pl.BlockSpec((row_blk, chunk), lambda r, k: (r, k))],
        out_specs=pl.BlockSpec((row_blk, 1), lambda r, k: (r, 0)),
        out_shape=jax.ShapeDtypeStruct((batch, 1), jnp.int32),
        scratch_shapes=[
            pltpu.VMEM((row_blk, width), jnp.float32),
            pltpu.VMEM((row_blk, width), jnp.int32),
        ],
        compiler_params=pltpu.CompilerParams(
            dimension_semantics=("parallel", "arbitrary"),
        ),
    )(logits)
    return out.reshape(batch)


# speedup vs baseline: 1.7684x; 1.0206x over previous
import functools

import jax
import jax.numpy as jnp
import numpy as np
from jax.experimental import pallas as pl
from jax.experimental.pallas import tpu as pltpu

_TINY = np.float32(1.1754943508222875e-38)


def _rotl(x, r):
    return (x << jnp.uint32(r)) | (x >> jnp.uint32(32 - r))


def _round4(x0, x1, rots):
    for r in rots:
        x0 = x0 + x1
        x1 = _rotl(x1, r) ^ x0
    return x0, x1


def _threefry_bits_pre(x1):
    ks1 = jnp.uint32(42)
    ks2 = jnp.uint32(0x1BD11BDA ^ 42)
    rot_a = (13, 15, 26, 6)
    rot_b = (17, 29, 16, 24)

    x0 = x1
    x1 = _rotl(x1, 13) ^ x0
    x0, x1 = _round4(x0, x1, rot_a[1:])
    x0 = x0 + ks1
    x1 = x1 + (ks2 + jnp.uint32(1))
    x0, x1 = _round4(x0, x1, rot_b)
    x0 = x0 + ks2
    x1 = x1 + jnp.uint32(2)
    x0, x1 = _round4(x0, x1, rot_a)
    x1 = x1 + (ks1 + jnp.uint32(3))
    x0, x1 = _round4(x0, x1, rot_b)
    x0 = x0 + ks1
    x1 = x1 + (ks2 + jnp.uint32(4))
    x0, x1 = _round4(x0, x1, rot_a)
    x0 = x0 + ks2
    x1 = x1 + jnp.uint32(5)
    return x0 ^ x1


def _body(vocab, n_chunks, width, lg_ref, out_ref, mx_ref, ix_ref):
    k = pl.program_id(1)
    rblk = pl.program_id(0)
    nb, chunk = lg_ref.shape
    n_sub = chunk // width

    last = k == n_chunks - 1
    limit = jnp.where(last, jnp.int32(vocab), jnp.int32(0x7FFFFFFF))

    row = jax.lax.broadcasted_iota(jnp.uint32, (nb, width), 0)
    colw = jax.lax.broadcasted_iota(jnp.uint32, (nb, width), 1)
    row0 = jnp.uint32(rblk) * jnp.uint32(nb)
    rowterm42 = row0 * jnp.uint32(vocab) + (row * jnp.uint32(vocab) + jnp.uint32(42))
    chunk0 = jnp.uint32(k) * jnp.uint32(chunk)

    def _sub(i, carry):
        mx, ix = carry
        base = chunk0 + jnp.uint32(i) * jnp.uint32(width)
        gcol_u = colw + base
        bits = _threefry_bits_pre(rowterm42 + gcol_u)
        fb = (bits >> jnp.uint32(9)) | jnp.uint32(0x3F800000)
        u = jax.lax.bitcast_convert_type(fb, jnp.float32) - jnp.float32(1.0)
        uni = jnp.maximum(u, _TINY)
        s = lg_ref[:, pl.ds(i * width, width)] - jnp.log(-jnp.log(uni))
        gcol = gcol_u.astype(jnp.int32)
        s = jnp.where(gcol < limit, s, -jnp.inf)
        better = s > mx
        mx = jnp.where(better, s, mx)
        ix = jnp.where(better, gcol, ix)
        return mx, ix

    mx0 = jnp.full((nb, width), -jnp.inf, jnp.float32)
    ix0 = jnp.zeros((nb, width), jnp.int32)
    mx, ix = jax.lax.fori_loop(0, n_sub, _sub, (mx0, ix0), unroll=32)

    @pl.when(k == 0)
    def _init():
        mx_ref[...] = mx
        ix_ref[...] = ix

    @pl.when(k > 0)
    def _merge():
        pmx = mx_ref[...]
        better = mx > pmx
        mx_ref[...] = jnp.where(better, mx, pmx)
        ix_ref[...] = jnp.where(better, ix, ix_ref[...])

    @pl.when(last)
    def _emit():
        fmx = mx_ref[...]
        m = jnp.max(fmx, axis=1, keepdims=True)
        cand = jnp.where(fmx == m, ix_ref[...], jnp.int32(0x7FFFFFFF))
        out_ref[...] = jnp.min(cand, axis=1, keepdims=True)


def kernel(logits):
    batch, vocab = logits.shape
    row_blk = 16
    chunk = 32768
    width = 512
    n_rblk = pl.cdiv(batch, row_blk)
    n_chunks = pl.cdiv(vocab, chunk)

    out = pl.pallas_call(
        functools.partial(_body, vocab, n_chunks, width),
        grid=(n_rblk, n_chunks),
        in_specs=[pl.BlockSpec((row_blk, chunk), lambda r, k: (r, k))],
        out_specs=pl.BlockSpec((row_blk, 1), lambda r, k: (r, 0)),
        out_shape=jax.ShapeDtypeStruct((batch, 1), jnp.int32),
        scratch_shapes=[
            pltpu.VMEM((row_blk, width), jnp.float32),
            pltpu.VMEM((row_blk, width), jnp.int32),
        ],
        compiler_params=pltpu.CompilerParams(
            dimension_semantics=("parallel", "arbitrary"),
        ),
    )(logits)
    return out.reshape(batch)


# speedup vs baseline: 1.7705x; 1.0012x over previous
import functools

import jax
import jax.numpy as jnp
import numpy as np
from jax.experimental import pallas as pl
from jax.experimental.pallas import tpu as pltpu

_TINY = np.float32(1.1754943508222875e-38)


def _rotl(x, r):
    return (x << jnp.uint32(r)) | (x >> jnp.uint32(32 - r))


def _round4(x0, x1, rots):
    for r in rots:
        x0 = x0 + x1
        x1 = _rotl(x1, r) ^ x0
    return x0, x1


def _threefry_bits_pre(x1):
    ks1 = jnp.uint32(42)
    ks2 = jnp.uint32(0x1BD11BDA ^ 42)
    rot_a = (13, 15, 26, 6)
    rot_b = (17, 29, 16, 24)

    x0 = x1
    x1 = _rotl(x1, 13) ^ x0
    x0, x1 = _round4(x0, x1, rot_a[1:])
    x0 = x0 + ks1
    x1 = x1 + (ks2 + jnp.uint32(1))
    x0, x1 = _round4(x0, x1, rot_b)
    x0 = x0 + ks2
    x1 = x1 + jnp.uint32(2)
    x0, x1 = _round4(x0, x1, rot_a)
    x1 = x1 + (ks1 + jnp.uint32(3))
    x0, x1 = _round4(x0, x1, rot_b)
    x0 = x0 + ks1
    x1 = x1 + (ks2 + jnp.uint32(4))
    x0, x1 = _round4(x0, x1, rot_a)
    x0 = x0 + ks2
    x1 = x1 + jnp.uint32(5)
    return x0 ^ x1


def _body(vocab, n_chunks, width, lg_ref, out_ref, mx_ref, ix_ref):
    k = pl.program_id(1)
    rblk = pl.program_id(0)
    nb, chunk = lg_ref.shape
    n_sub = chunk // width

    last = k == n_chunks - 1
    limit = jnp.where(last, jnp.int32(vocab), jnp.int32(0x7FFFFFFF))

    row = jax.lax.broadcasted_iota(jnp.uint32, (nb, width), 0)
    colw = jax.lax.broadcasted_iota(jnp.uint32, (nb, width), 1)
    row0 = jnp.uint32(rblk) * jnp.uint32(nb)
    rowterm42 = row0 * jnp.uint32(vocab) + (row * jnp.uint32(vocab) + jnp.uint32(42))
    chunk0 = jnp.uint32(k) * jnp.uint32(chunk)

    def _sub(i, carry):
        mx, ix = carry
        base = chunk0 + jnp.uint32(i) * jnp.uint32(width)
        gcol_u = colw + base
        bits = _threefry_bits_pre(rowterm42 + gcol_u)
        fb = (bits >> jnp.uint32(9)) | jnp.uint32(0x3F800000)
        u = jax.lax.bitcast_convert_type(fb, jnp.float32) - jnp.float32(1.0)
        uni = jnp.maximum(u, _TINY)
        s = lg_ref[:, pl.ds(i * width, width)] - jnp.log(-jnp.log(uni))
        gcol = gcol_u.astype(jnp.int32)
        s = jnp.where(gcol < limit, s, -jnp.inf)
        better = s > mx
        mx = jnp.where(better, s, mx)
        ix = jnp.where(better, gcol, ix)
        return mx, ix

    mx0 = jnp.full((nb, width), -jnp.inf, jnp.float32)
    ix0 = jnp.zeros((nb, width), jnp.int32)
    mx, ix = jax.lax.fori_loop(0, n_sub, _sub, (mx0, ix0), unroll=64)

    @pl.when(k == 0)
    def _init():
        mx_ref[...] = mx
        ix_ref[...] = ix

    @pl.when(k > 0)
    def _merge():
        pmx = mx_ref[...]
        better = mx > pmx
        mx_ref[...] = jnp.where(better, mx, pmx)
        ix_ref[...] = jnp.where(better, ix, ix_ref[...])

    @pl.when(last)
    def _emit():
        fmx = mx_ref[...]
        m = jnp.max(fmx, axis=1, keepdims=True)
        cand = jnp.where(fmx == m, ix_ref[...], jnp.int32(0x7FFFFFFF))
        out_ref[...] = jnp.min(cand, axis=1, keepdims=True)


def kernel(logits):
    batch, vocab = logits.shape
    row_blk = 16
    chunk = 32768
    width = 256
    n_rblk = pl.cdiv(batch, row_blk)
    n_chunks = pl.cdiv(vocab, chunk)

    out = pl.pallas_call(
        functools.partial(_body, vocab, n_chunks, width),
        grid=(n_rblk, n_chunks),
        in_specs=[pl.BlockSpec((row_blk, chunk), lambda r, k: (r, k))],
        out_specs=pl.BlockSpec((row_blk, 1), lambda r, k: (r, 0)),
        out_shape=jax.ShapeDtypeStruct((batch, 1), jnp.int32),
        scratch_shapes=[
            pltpu.VMEM((row_blk, width), jnp.float32),
            pltpu.VMEM((row_blk, width), jnp.int32),
        ],
        compiler_params=pltpu.CompilerParams(
            dimension_semantics=("parallel", "arbitrary"),
        ),
    )(logits)
    return out.reshape(batch)
